# baseline (speedup 1.0000x reference)
import functools

import jax
import jax.numpy as jnp
from jax.experimental import pallas as pl
from jax.experimental.pallas import tpu as pltpu

D_MODEL = 1024
D_FF = 4096
N_EXP = 8
TOKENS = 2048

T_BLK = 256
F_BLK = 512


def _moe_body(x_ref, wg_ref, w1_ref, b1_ref, w2_ref, b2_ref, out_ref, w8_ref):
    e = pl.program_id(1)
    f = pl.program_id(2)

    @pl.when(jnp.logical_and(e == 0, f == 0))
    def _gate():
        x = x_ref[...]
        logits = jax.lax.dot_general(
            x, wg_ref[...], (((1,), (0,)), ((), ())),
            preferred_element_type=jnp.float32)
        g = jax.nn.softmax(logits, axis=-1)
        iota = jax.lax.broadcasted_iota(jnp.int32, g.shape, 1)
        i1 = jnp.argmax(g, axis=-1)[:, None]
        m1 = jnp.max(g, axis=-1, keepdims=True)
        gm = jnp.where(iota == i1, -1.0, g)
        i2 = jnp.argmax(gm, axis=-1)[:, None]
        m2 = jnp.max(gm, axis=-1, keepdims=True)
        s = m1 + m2
        w8 = jnp.where(iota == i1, m1 / s, 0.0) + jnp.where(iota == i2, m2 / s, 0.0)
        w8_ref[...] = w8

    x = x_ref[...]
    w8 = w8_ref[...]
    iota_e = jax.lax.broadcasted_iota(jnp.int32, w8.shape, 1)
    w_col = jnp.sum(jnp.where(iota_e == e, w8, 0.0), axis=1, keepdims=True)
    h = jax.lax.dot_general(
        x, w1_ref[0], (((1,), (0,)), ((), ())),
        preferred_element_type=jnp.float32) + b1_ref[0]
    h = jax.nn.gelu(h)
    contrib = jax.lax.dot_general(
        h, w2_ref[0], (((1,), (0,)), ((), ())),
        preferred_element_type=jnp.float32)
    contrib = w_col * contrib

    @pl.when(f == 0)
    def _bias():
        contrib_b = contrib + w_col * b2_ref[0]

        @pl.when(e == 0)
        def _init():
            out_ref[...] = contrib_b

        @pl.when(e != 0)
        def _acc():
            out_ref[...] += contrib_b

    @pl.when(f != 0)
    def _acc2():
        out_ref[...] += contrib


@functools.partial(jax.jit, static_argnames=())
def kernel(inputs, Wg, W1, b1, W2, b2):
    flat = inputs.reshape((-1, inputs.shape[-1]))
    T = flat.shape[0]
    grid = (T // T_BLK, N_EXP, D_FF // F_BLK)
    out = pl.pallas_call(
        _moe_body,
        grid=grid,
        in_specs=[
            pl.BlockSpec((T_BLK, D_MODEL), lambda t, e, f: (t, 0)),
            pl.BlockSpec((D_MODEL, N_EXP), lambda t, e, f: (0, 0)),
            pl.BlockSpec((1, D_MODEL, F_BLK), lambda t, e, f: (e, 0, f)),
            pl.BlockSpec((1, 1, F_BLK), lambda t, e, f: (e, 0, f)),
            pl.BlockSpec((1, F_BLK, D_MODEL), lambda t, e, f: (e, f, 0)),
            pl.BlockSpec((1, 1, D_MODEL), lambda t, e, f: (e, 0, 0)),
        ],
        out_specs=pl.BlockSpec((T_BLK, D_MODEL), lambda t, e, f: (t, 0)),
        out_shape=jax.ShapeDtypeStruct((T, D_MODEL), jnp.float32),
        scratch_shapes=[pltpu.VMEM((T_BLK, N_EXP), jnp.float32)],
    )(flat, Wg, W1, b1.reshape(N_EXP, 1, D_FF), W2, b2.reshape(N_EXP, 1, D_MODEL))
    return out.reshape(inputs.shape)


# speedup vs baseline: 1.6463x; 1.6463x over previous
import functools

import jax
import jax.numpy as jnp
from jax.experimental import pallas as pl
from jax.experimental.pallas import tpu as pltpu

D_MODEL = 1024
D_FF = 4096
N_EXP = 8
TOKENS = 2048

T_BLK = 2048
F_BLK = 512


def _moe_body(x_ref, xb_ref, wg_ref, w1_ref, b1_ref, w2_ref, b2_ref,
              out_ref, w8_ref):
    e = pl.program_id(0)
    f = pl.program_id(1)

    @pl.when(jnp.logical_and(e == 0, f == 0))
    def _gate():
        x = x_ref[...]
        logits = jax.lax.dot_general(
            x, wg_ref[...], (((1,), (0,)), ((), ())),
            preferred_element_type=jnp.float32)
        g = jax.nn.softmax(logits, axis=-1)
        iota = jax.lax.broadcasted_iota(jnp.int32, g.shape, 1)
        i1 = jnp.argmax(g, axis=-1)[:, None]
        m1 = jnp.max(g, axis=-1, keepdims=True)
        gm = jnp.where(iota == i1, -1.0, g)
        i2 = jnp.argmax(gm, axis=-1)[:, None]
        m2 = jnp.max(gm, axis=-1, keepdims=True)
        s = m1 + m2
        w8_ref[...] = (jnp.where(iota == i1, m1 / s, 0.0)
                       + jnp.where(iota == i2, m2 / s, 0.0))

    w8 = w8_ref[...]
    iota_e = jax.lax.broadcasted_iota(jnp.int32, w8.shape, 1)
    w_col = jnp.sum(jnp.where(iota_e == e, w8, 0.0), axis=1, keepdims=True)

    h = jax.lax.dot_general(
        xb_ref[...], w1_ref[0], (((1,), (0,)), ((), ())),
        preferred_element_type=jnp.float32) + b1_ref[0]
    h = jax.nn.gelu(h).astype(jnp.bfloat16)
    contrib = jax.lax.dot_general(
        h, w2_ref[0], (((1,), (0,)), ((), ())),
        preferred_element_type=jnp.float32)
    contrib = w_col * contrib

    @pl.when(f == 0)
    def _bias():
        contrib_b = contrib + w_col * b2_ref[0]

        @pl.when(e == 0)
        def _init():
            out_ref[...] = contrib_b

        @pl.when(e != 0)
        def _acc():
            out_ref[...] += contrib_b

    @pl.when(f != 0)
    def _acc2():
        out_ref[...] += contrib


def kernel(inputs, Wg, W1, b1, W2, b2):
    flat = inputs.reshape((-1, inputs.shape[-1]))
    T = flat.shape[0]
    flat_b = flat.astype(jnp.bfloat16)
    W1b = W1.astype(jnp.bfloat16)
    W2b = W2.astype(jnp.bfloat16)
    grid = (N_EXP, D_FF // F_BLK)
    out = pl.pallas_call(
        _moe_body,
        grid=grid,
        in_specs=[
            pl.BlockSpec((T_BLK, D_MODEL), lambda e, f: (0, 0)),
            pl.BlockSpec((T_BLK, D_MODEL), lambda e, f: (0, 0)),
            pl.BlockSpec((D_MODEL, N_EXP), lambda e, f: (0, 0)),
            pl.BlockSpec((1, D_MODEL, F_BLK), lambda e, f: (e, 0, f)),
            pl.BlockSpec((1, 1, F_BLK), lambda e, f: (e, 0, f)),
            pl.BlockSpec((1, F_BLK, D_MODEL), lambda e, f: (e, f, 0)),
            pl.BlockSpec((1, 1, D_MODEL), lambda e, f: (e, 0, 0)),
        ],
        out_specs=pl.BlockSpec((T_BLK, D_MODEL), lambda e, f: (0, 0)),
        out_shape=jax.ShapeDtypeStruct((T, D_MODEL), jnp.float32),
        scratch_shapes=[pltpu.VMEM((T_BLK, N_EXP), jnp.float32)],
    )(flat, flat_b, Wg, W1b, b1.reshape(N_EXP, 1, D_FF), W2b,
      b2.reshape(N_EXP, 1, D_MODEL))
    return out.reshape(inputs.shape)
